# baseline (speedup 1.0000x reference)
import functools

import jax
import jax.numpy as jnp
from jax import lax
from jax.experimental import pallas as pl
from jax.experimental.pallas import tpu as pltpu
from jax.experimental.pallas import tpu_sc as plsc

N = 100000
D = 64
OUT_D = 3 * D + 1
CB = 80
NUM_CHUNKS = N // CB
NC = 2
NS = 16
L = 16
NW = NC * NS
CPW = -(-NUM_CHUNKS // NW)

_LN2 = 0.6931471805599453
_SQRT2 = 1.4142135623730951


def _log1p_16(x):
    y = jnp.maximum(x, 0.0) + 1.0
    yi = lax.bitcast_convert_type(y, jnp.int32)
    e = lax.shift_right_logical(yi, 23) - 127
    m = lax.bitcast_convert_type(
        (yi & 0x007FFFFF) | 0x3F800000, jnp.float32)
    big = m > _SQRT2
    m = jnp.where(big, m * 0.5, m)
    ef = (e + big.astype(jnp.int32)).astype(jnp.float32)
    s = (m - 1.0) / (m + 1.0)
    z = s * s
    p = 2.0 * s * (1.0 + z * (1.0 / 3.0 + z * (1.0 / 5.0 + z * (1.0 / 7.0 + z * (1.0 / 9.0)))))
    return ef * _LN2 + p


_mesh = plsc.VectorSubcoreMesh(core_axis_name="c", subcore_axis_name="s")


def _scratch_set():
    return [
        pltpu.VMEM((CB,), jnp.int32),
        pltpu.VMEM((CB,), jnp.int32),
        pltpu.VMEM((CB,), jnp.int32),
        pltpu.VMEM((CB,), jnp.float32),
        pltpu.VMEM((CB,), jnp.int32),
        pltpu.VMEM((CB, 2 * D), jnp.float32),
        pltpu.VMEM((CB, 2 * D), jnp.float32),
        pltpu.VMEM((CB, OUT_D), jnp.float32),
        pltpu.SemaphoreType.DMA,
        pltpu.SemaphoreType.DMA,
        pltpu.SemaphoreType.DMA,
    ]


@functools.partial(
    pl.kernel,
    out_type=jax.ShapeDtypeStruct((N, OUT_D), jnp.float32),
    mesh=_mesh,
    scratch_types=_scratch_set() + _scratch_set(),
    compiler_params=pltpu.CompilerParams(needs_layout_passes=False),
)
def _embed(op_id_hbm, svc_id_hbm, st_id_hbm, lat_hbm,
           op2_hbm, combo_hbm, out_hbm, *bufs):
    set0, set1 = bufs[:11], bufs[11:]
    wid = lax.axis_index("s") * NC + lax.axis_index("c")

    def fire_idx(cid, b):
        base = cid * CB
        pltpu.async_copy(op_id_hbm.at[pl.ds(base, CB)], b[0], b[8])
        pltpu.async_copy(svc_id_hbm.at[pl.ds(base, CB)], b[1], b[8])
        pltpu.async_copy(st_id_hbm.at[pl.ds(base, CB)], b[2], b[8])
        pltpu.async_copy(lat_hbm.at[pl.ds(base, CB)], b[3], b[8])

    def wait_idx(b):
        pltpu.make_async_copy(op_id_hbm.at[pl.ds(0, CB)], b[0], b[8]).wait()
        pltpu.make_async_copy(svc_id_hbm.at[pl.ds(0, CB)], b[1], b[8]).wait()
        pltpu.make_async_copy(st_id_hbm.at[pl.ds(0, CB)], b[2], b[8]).wait()
        pltpu.make_async_copy(lat_hbm.at[pl.ds(0, CB)], b[3], b[8]).wait()

    def fire_gathers(b):
        opid, svcid, stid, par = b[0], b[1], b[2], b[4]
        a_v, comb_v, semg = b[5], b[6], b[9]

        @pl.loop(0, CB // L)
        def _prep(j):
            sl = pl.ds(j * L, L)
            stid[sl] = svcid[sl] * 16 + stid[sl]
            o = opid[sl]
            par[sl] = (o & 1) * D
            opid[sl] = lax.shift_right_logical(o, 1)

        pltpu.async_copy(combo_hbm.at[stid], comb_v, semg)
        pltpu.async_copy(op2_hbm.at[opid], a_v, semg)

    def wait_gathers(b):
        pltpu.make_async_copy(combo_hbm.at[pl.ds(0, CB)], b[6], b[9]).wait()
        pltpu.make_async_copy(op2_hbm.at[pl.ds(0, CB)], b[5], b[9]).wait()

    def wait_out(b):
        pltpu.make_async_copy(b[7], out_hbm.at[pl.ds(0, CB)], b[10]).wait()

    def assemble_fire_out(cid, b):
        latv, par, a_v, comb_v, rows_v = b[3], b[4], b[5], b[6], b[7]

        @pl.loop(0, CB // L)
        def _lat(j):
            sl = pl.ds(j * L, L)
            latv[sl] = _log1p_16(latv[sl])

        @pl.loop(0, CB // L)
        def _rows(g):
            pvec = par[pl.ds(g * L, L)]
            for j in range(L):
                r = g * L + j
                off = pvec[j]
                for t in range(D // L):
                    rows_v[r, pl.ds(t * L, L)] = a_v[r, pl.ds(off + t * L, L)]
                for t in range(2 * D // L):
                    rows_v[r, pl.ds(D + t * L, L)] = comb_v[r, pl.ds(t * L, L)]

        @pl.loop(0, CB // L)
        def _latcol(j):
            f = latv[pl.ds(j * L, L)]
            ridx = lax.iota(jnp.int32, L) + j * L
            cidx = jnp.full((L,), 3 * D, jnp.int32)
            plsc.store_scatter(rows_v, [ridx, cidx], f)

        pltpu.async_copy(rows_v, out_hbm.at[pl.ds(cid * CB, CB)], b[10])

    def phase(i_expr, cur, nxt, not_first):
        cid_cur = wid + i_expr * NW
        cid_nxt = cid_cur + NW

        @pl.when(cid_nxt < NUM_CHUNKS)
        def _():
            fire_idx(cid_nxt, nxt)

        @pl.when(cid_cur < NUM_CHUNKS)
        def _():
            wait_gathers(cur)

        @pl.when(cid_nxt < NUM_CHUNKS)
        def _():
            wait_idx(nxt)
            fire_gathers(nxt)

        @pl.when(cid_cur < NUM_CHUNKS)
        def _():
            @pl.when(not_first)
            def _():
                wait_out(cur)
            assemble_fire_out(cid_cur, cur)

    fire_idx(wid, set0)
    wait_idx(set0)
    fire_gathers(set0)

    @pl.loop(0, CPW // 2)
    def _main(g2):
        i0 = 2 * g2
        phase(i0, set0, set1, g2 >= 1)
        phase(i0 + 1, set1, set0, g2 >= 1)

    wait_out(set0)
    wait_out(set1)


def kernel(operation_id, service_id, status_id, latency, op_table, svc_table, status_table):
    combo = jnp.concatenate([
        jnp.broadcast_to(svc_table[:, None, :], (svc_table.shape[0], 16, D)),
        jnp.broadcast_to(status_table[None, :, :], (svc_table.shape[0], 16, D)),
    ], axis=-1).reshape(svc_table.shape[0] * 16, 2 * D)
    op2 = op_table.reshape(op_table.shape[0] // 2, 2 * D)
    return _embed(operation_id.astype(jnp.int32), service_id.astype(jnp.int32),
                  status_id.astype(jnp.int32), latency,
                  op2, combo)


# speedup vs baseline: 1.4144x; 1.4144x over previous
import functools

import jax
import jax.numpy as jnp
from jax import lax
from jax.experimental import pallas as pl
from jax.experimental.pallas import tpu as pltpu
from jax.experimental.pallas import tpu_sc as plsc

N = 100000
D = 64
OUT_D = 3 * D + 1
CB = 80
K = 16
NUM_CHUNKS = N // CB
NC = 2
NS = 16
L = 16
NW = NC * NS
CPW = -(-NUM_CHUNKS // NW)
CPW3 = ((CPW + 3) // 3) * 3

_LN2 = 0.6931471805599453
_SQRT2 = 1.4142135623730951


def _log1p_16(x):
    y = jnp.maximum(x, 0.0) + 1.0
    yi = lax.bitcast_convert_type(y, jnp.int32)
    e = lax.shift_right_logical(yi, 23) - 127
    m = lax.bitcast_convert_type(
        (yi & 0x007FFFFF) | 0x3F800000, jnp.float32)
    big = m > _SQRT2
    m = jnp.where(big, m * 0.5, m)
    ef = (e + big.astype(jnp.int32)).astype(jnp.float32)
    s = (m - 1.0) / (m + 1.0)
    z = s * s
    p = 2.0 * s * (1.0 + z * (1.0 / 3.0 + z * (1.0 / 5.0 + z * (1.0 / 7.0 + z * (1.0 / 9.0)))))
    return ef * _LN2 + p


_mesh = plsc.VectorSubcoreMesh(core_axis_name="c", subcore_axis_name="s")


def _scratch_set():
    return [
        pltpu.VMEM((CB,), jnp.int32),
        pltpu.VMEM((CB,), jnp.int32),
        pltpu.VMEM((CB,), jnp.int32),
        pltpu.VMEM((CB,), jnp.float32),
        pltpu.VMEM((CB, D), jnp.float32),
        pltpu.VMEM((CB, 2 * D), jnp.float32),
        pltpu.VMEM((CB, OUT_D), jnp.float32),
        pltpu.SemaphoreType.DMA,
        pltpu.SemaphoreType.DMA,
        pltpu.SemaphoreType.DMA,
    ]


@functools.partial(
    pl.kernel,
    out_type=jax.ShapeDtypeStruct((N, OUT_D), jnp.float32),
    mesh=_mesh,
    scratch_types=_scratch_set() + _scratch_set() + _scratch_set(),
    compiler_params=pltpu.CompilerParams(needs_layout_passes=False),
)
def _embed(op_id_hbm, svc_id_hbm, st_id_hbm, lat_hbm,
           op_tab_hbm, combo_hbm, out_hbm, *bufs):
    sets = (bufs[:10], bufs[10:20], bufs[20:])
    wid = lax.axis_index("s") * NC + lax.axis_index("c")

    def fire_idx(cid, b):
        base = cid * CB
        pltpu.async_copy(op_id_hbm.at[pl.ds(base, CB)], b[0], b[7])
        pltpu.async_copy(svc_id_hbm.at[pl.ds(base, CB)], b[1], b[7])
        pltpu.async_copy(st_id_hbm.at[pl.ds(base, CB)], b[2], b[7])
        pltpu.async_copy(lat_hbm.at[pl.ds(base, CB)], b[3], b[7])

    def wait_idx(b):
        pltpu.make_async_copy(op_id_hbm.at[pl.ds(0, CB)], b[0], b[7]).wait()
        pltpu.make_async_copy(svc_id_hbm.at[pl.ds(0, CB)], b[1], b[7]).wait()
        pltpu.make_async_copy(st_id_hbm.at[pl.ds(0, CB)], b[2], b[7]).wait()
        pltpu.make_async_copy(lat_hbm.at[pl.ds(0, CB)], b[3], b[7]).wait()

    def fire_gathers(b):
        opid, svcid, stid = b[0], b[1], b[2]
        a_v, comb_v, semg = b[4], b[5], b[8]

        @pl.loop(0, CB // L)
        def _cidx(j):
            sl = pl.ds(j * L, L)
            stid[sl] = svcid[sl] * 16 + stid[sl]

        pltpu.async_copy(combo_hbm.at[stid], comb_v, semg)

        @pl.loop(0, CB // K)
        def _oprows(g):
            vec = opid[pl.ds(g * K, K)]
            for j in range(K):
                pltpu.async_copy(
                    op_tab_hbm.at[pl.ds(vec[j], 1)],
                    a_v.at[pl.ds(g * K + j, 1)], semg)

    def wait_gathers(b):
        pltpu.make_async_copy(combo_hbm.at[pl.ds(0, CB)], b[5], b[8]).wait()
        pltpu.make_async_copy(op_tab_hbm.at[pl.ds(0, CB)], b[4], b[8]).wait()

    def wait_out(b):
        pltpu.make_async_copy(b[6], out_hbm.at[pl.ds(0, CB)], b[9]).wait()

    def assemble_fire_out(cid, b):
        latv, a_v, comb_v, rows_v = b[3], b[4], b[5], b[6]

        @pl.loop(0, CB // L)
        def _lat(j):
            sl = pl.ds(j * L, L)
            latv[sl] = _log1p_16(latv[sl])

        @pl.loop(0, CB)
        def _row(r):
            for t in range(D // L):
                rows_v[r, pl.ds(t * L, L)] = a_v[r, pl.ds(t * L, L)]
            for t in range(2 * D // L):
                rows_v[r, pl.ds(D + t * L, L)] = comb_v[r, pl.ds(t * L, L)]

        @pl.loop(0, CB // L)
        def _latcol(j):
            f = latv[pl.ds(j * L, L)]
            ridx = lax.iota(jnp.int32, L) + j * L
            cidx = jnp.full((L,), 3 * D, jnp.int32)
            plsc.store_scatter(rows_v, [ridx, cidx], f)

        pltpu.async_copy(rows_v, out_hbm.at[pl.ds(cid * CB, CB)], b[9])

    def phase(i_expr, cur, nx1, nx2, past_start):
        cid_cur = wid + i_expr * NW
        cid_nx1 = cid_cur + NW
        cid_nx2 = cid_cur + 2 * NW

        @pl.when(cid_nx2 < NUM_CHUNKS)
        def _():
            fire_idx(cid_nx2, nx2)

        @pl.when(cid_nx1 < NUM_CHUNKS)
        def _():
            wait_idx(nx1)
            fire_gathers(nx1)

        @pl.when(cid_cur < NUM_CHUNKS)
        def _():
            wait_gathers(cur)

            @pl.when(past_start)
            def _():
                wait_out(cur)

            assemble_fire_out(cid_cur, cur)

    fire_idx(wid, sets[0])
    fire_idx(wid + NW, sets[1])
    wait_idx(sets[0])
    fire_gathers(sets[0])

    @pl.loop(0, CPW3 // 3)
    def _main(g3):
        i0 = 3 * g3
        phase(i0, sets[0], sets[1], sets[2], g3 >= 1)
        phase(i0 + 1, sets[1], sets[2], sets[0], i0 + 1 >= 3)
        phase(i0 + 2, sets[2], sets[0], sets[1], i0 + 2 >= 3)

    wait_out(sets[0])
    wait_out(sets[1])
    wait_out(sets[2])


def kernel(operation_id, service_id, status_id, latency, op_table, svc_table, status_table):
    combo = jnp.concatenate([
        jnp.broadcast_to(svc_table[:, None, :], (svc_table.shape[0], 16, D)),
        jnp.broadcast_to(status_table[None, :, :], (svc_table.shape[0], 16, D)),
    ], axis=-1).reshape(svc_table.shape[0] * 16, 2 * D)
    return _embed(operation_id.astype(jnp.int32), service_id.astype(jnp.int32),
                  status_id.astype(jnp.int32), latency,
                  op_table, combo)


# speedup vs baseline: 1.8007x; 1.2731x over previous
import functools

import jax
import jax.numpy as jnp
from jax import lax
from jax.experimental import pallas as pl
from jax.experimental.pallas import tpu as pltpu
from jax.experimental.pallas import tpu_sc as plsc

N = 100000
D = 64
OUT_D = 3 * D + 1
CB = 80
K = 16
NUM_CHUNKS = N // CB
NC = 2
NS = 16
L = 16
NW = NC * NS
CPW = -(-NUM_CHUNKS // NW)
CPW3 = ((CPW + 3) // 3) * 3

_LN2 = 0.6931471805599453
_SQRT2 = 1.4142135623730951


def _log1p_16(x):
    y = jnp.maximum(x, 0.0) + 1.0
    yi = lax.bitcast_convert_type(y, jnp.int32)
    e = lax.shift_right_logical(yi, 23) - 127
    m = lax.bitcast_convert_type(
        (yi & 0x007FFFFF) | 0x3F800000, jnp.float32)
    big = m > _SQRT2
    m = jnp.where(big, m * 0.5, m)
    ef = (e + big.astype(jnp.int32)).astype(jnp.float32)
    s = (m - 1.0) / (m + 1.0)
    z = s * s
    p = 2.0 * s * (1.0 + z * (1.0 / 3.0 + z * (1.0 / 5.0 + z * (1.0 / 7.0 + z * (1.0 / 9.0)))))
    return ef * _LN2 + p


_mesh = plsc.VectorSubcoreMesh(core_axis_name="c", subcore_axis_name="s")


def _scratch_set():
    return [
        pltpu.VMEM((CB,), jnp.int32),
        pltpu.VMEM((CB,), jnp.int32),
        pltpu.VMEM((CB,), jnp.int32),
        pltpu.VMEM((CB,), jnp.float32),
        pltpu.VMEM((CB, D), jnp.float32),
        pltpu.VMEM((CB, 2 * D), jnp.float32),
        pltpu.VMEM((CB, OUT_D), jnp.float32),
        pltpu.SemaphoreType.DMA,
        pltpu.SemaphoreType.DMA,
        pltpu.SemaphoreType.DMA,
    ]


@functools.partial(
    pl.kernel,
    out_type=jax.ShapeDtypeStruct((N, OUT_D), jnp.float32),
    mesh=_mesh,
    scratch_types=_scratch_set() + _scratch_set() + _scratch_set(),
    compiler_params=pltpu.CompilerParams(needs_layout_passes=False),
)
def _embed(op_id_hbm, svc_id_hbm, st_id_hbm, lat_hbm,
           op_tab_hbm, combo_hbm, out_hbm, *bufs):
    sets = (bufs[:10], bufs[10:20], bufs[20:])
    wid = lax.axis_index("s") * NC + lax.axis_index("c")

    def fire_idx(cid, b):
        base = cid * CB
        pltpu.async_copy(op_id_hbm.at[pl.ds(base, CB)], b[0], b[7])
        pltpu.async_copy(svc_id_hbm.at[pl.ds(base, CB)], b[1], b[7])
        pltpu.async_copy(st_id_hbm.at[pl.ds(base, CB)], b[2], b[7])
        pltpu.async_copy(lat_hbm.at[pl.ds(base, CB)], b[3], b[7])

    def wait_idx(b):
        pltpu.make_async_copy(op_id_hbm.at[pl.ds(0, CB)], b[0], b[7]).wait()
        pltpu.make_async_copy(svc_id_hbm.at[pl.ds(0, CB)], b[1], b[7]).wait()
        pltpu.make_async_copy(st_id_hbm.at[pl.ds(0, CB)], b[2], b[7]).wait()
        pltpu.make_async_copy(lat_hbm.at[pl.ds(0, CB)], b[3], b[7]).wait()

    def fire_gathers(b):
        opid, svcid, stid = b[0], b[1], b[2]
        a_v, comb_v, semg = b[4], b[5], b[8]

        @pl.loop(0, CB // L)
        def _cidx(j):
            sl = pl.ds(j * L, L)
            stid[sl] = svcid[sl] * 16 + stid[sl]

        pltpu.async_copy(combo_hbm.at[stid], comb_v, semg)

        @pl.loop(0, CB // K)
        def _oprows(g):
            vec = opid[pl.ds(g * K, K)]
            for j in range(K):
                pltpu.async_copy(
                    op_tab_hbm.at[pl.ds(vec[j], 1)],
                    a_v.at[pl.ds(g * K + j, 1)], semg)

    def wait_gathers(b):
        pltpu.make_async_copy(combo_hbm.at[pl.ds(0, CB)], b[5], b[8]).wait()
        pltpu.make_async_copy(op_tab_hbm.at[pl.ds(0, CB)], b[4], b[8]).wait()

    def wait_out(b):
        pltpu.make_async_copy(b[6], out_hbm.at[pl.ds(0, CB)], b[9]).wait()

    def assemble_fire_out(cid, b):
        latv, a_v, comb_v, rows_v = b[3], b[4], b[5], b[6]

        @pl.loop(0, CB // L)
        def _lat(j):
            sl = pl.ds(j * L, L)
            latv[sl] = _log1p_16(latv[sl])

        @pl.loop(0, CB)
        def _row(r):
            for t in range(D // L):
                rows_v[r, pl.ds(t * L, L)] = a_v[r, pl.ds(t * L, L)]
            for t in range(2 * D // L):
                rows_v[r, pl.ds(D + t * L, L)] = comb_v[r, pl.ds(t * L, L)]

        @pl.loop(0, CB // L)
        def _latcol(j):
            f = latv[pl.ds(j * L, L)]
            ridx = lax.iota(jnp.int32, L) + j * L
            cidx = jnp.full((L,), 3 * D, jnp.int32)
            plsc.store_scatter(rows_v, [ridx, cidx], f)

        pltpu.async_copy(rows_v, out_hbm.at[pl.ds(cid * CB, CB)], b[9])

    def phase(i_expr, cur, nx1, nx2, past_start):
        cid_cur = wid + i_expr * NW
        cid_nx1 = cid_cur + NW
        cid_nx2 = cid_cur + 2 * NW

        @pl.when(cid_nx2 < NUM_CHUNKS)
        def _():
            fire_idx(cid_nx2, nx2)

        @pl.when(cid_nx1 < NUM_CHUNKS)
        def _():
            wait_idx(nx1)
            fire_gathers(nx1)

        @pl.when(cid_cur < NUM_CHUNKS)
        def _():
            wait_gathers(cur)

            @pl.when(past_start)
            def _():
                wait_out(cur)

            assemble_fire_out(cid_cur, cur)

    fire_idx(wid, sets[0])
    fire_idx(wid + NW, sets[1])
    wait_idx(sets[0])
    fire_gathers(sets[0])

    @pl.loop(0, CPW3 // 3)
    def _main(g3):
        i0 = 3 * g3
        phase(i0, sets[0], sets[1], sets[2], g3 >= 1)
        phase(i0 + 1, sets[1], sets[2], sets[0], i0 + 1 >= 3)
        phase(i0 + 2, sets[2], sets[0], sets[1], i0 + 2 >= 3)

    wait_out(sets[0])
    wait_out(sets[1])
    wait_out(sets[2])


def kernel(operation_id, service_id, status_id, latency, op_table, svc_table, status_table):
    combo = jnp.concatenate([
        jnp.broadcast_to(svc_table[:, None, :], (svc_table.shape[0], 16, D)),
        jnp.broadcast_to(status_table[None, :, :], (svc_table.shape[0], 16, D)),
    ], axis=-1).reshape(svc_table.shape[0] * 16, 2 * D)
    op_rm = lax.optimization_barrier(op_table.T).T
    return _embed(operation_id.astype(jnp.int32), service_id.astype(jnp.int32),
                  status_id.astype(jnp.int32), latency,
                  op_rm, combo)
